# baseline (speedup 1.0000x reference)
import functools

import jax
import jax.numpy as jnp
from jax import lax
from jax.experimental import pallas as pl
from jax.experimental.pallas import tpu as pltpu
from jax.experimental.pallas import tpu_sc as plsc

_BATCH = 16384
_DIM = 64
_NC = 2
_NS = 16
_NW = _NC * _NS
_BPW = _BATCH // _NW
_CH = 64
_NCH = _BPW // _CH
_L = 16
_GPC = _CH // _L


def _sc_body(uvec_hbm, ivec_hbm, uemb_hbm, iemb_hbm, out_hbm,
             uixv, iixv, ush, ish, ub, ib, outv, *sems):
    cid = lax.axis_index("c")
    sid = lax.axis_index("s")
    wid = sid * _NC + cid
    base = wid * _BPW

    pltpu.sync_copy(uvec_hbm.at[pl.ds(base, _BPW)], uixv)
    pltpu.sync_copy(ivec_hbm.at[pl.ds(base, _BPW)], iixv)

    usems = (sems[0], sems[1])
    isems = (sems[2], sems[3])

    def fire(c):
        par = c % 2

        def body(q, carry):
            v_u = uixv[pl.ds(c * _CH + q * _L, _L)]
            v_i = iixv[pl.ds(c * _CH + q * _L, _L)]
            for l in range(_L):
                t = c * _CH + q * _L + l
                dst = pl.ds(t * _DIM, _DIM)
                pltpu.async_copy(uemb_hbm.at[v_u[l]], ush.at[sid, dst],
                                 usems[par])
                pltpu.async_copy(iemb_hbm.at[v_i[l]], ish.at[sid, dst],
                                 isems[par])
            return carry

        lax.fori_loop(0, _GPC, body, 0)

    cbytes = _CH * _DIM

    def drain(c):
        par = c % 2
        dummy = out_hbm.at[pl.ds(0, cbytes)]
        pltpu.make_async_copy(
            dummy, ush.at[sid, pl.ds(0, cbytes)], usems[par]).wait()
        pltpu.make_async_copy(
            dummy, ish.at[sid, pl.ds(0, cbytes)], isems[par]).wait()

    iota = lax.iota(jnp.int32, _L)

    def compute(c):
        sl = pl.ds(c * _CH * _DIM, _CH * _DIM)
        pltpu.sync_copy(ush.at[sid, sl], ub)
        pltpu.sync_copy(ish.at[sid, sl], ib)
        for g in range(_GPC):
            rowbase = (g * _L + iota) * _DIM

            def body(d, acc):
                flat = rowbase + d
                uu = plsc.load_gather(ub, [flat])
                vv = plsc.load_gather(ib, [flat])
                return acc + uu * vv

            acc = lax.fori_loop(0, _DIM, body, jnp.zeros((_L,), jnp.float32))
            outv[pl.ds(c * _CH + g * _L, _L)] = 1.0 / (1.0 + jnp.exp(-acc))

    fire(0)
    for c in range(_NCH):
        if c + 1 < _NCH:
            fire(c + 1)
        drain(c)
        compute(c)

    pltpu.sync_copy(outv, out_hbm.at[pl.ds(base, _BPW)])


@jax.jit
def _run(users_vec, items_vec, users_emb, items_emb):
    k = functools.partial(
        pl.kernel,
        mesh=plsc.VectorSubcoreMesh(core_axis_name="c", subcore_axis_name="s"),
        out_type=jax.ShapeDtypeStruct((_BATCH,), jnp.float32),
        compiler_params=pltpu.CompilerParams(
            needs_layout_passes=False,
        ),
        scratch_types=[
            pltpu.VMEM((_BPW,), jnp.int32),
            pltpu.VMEM((_BPW,), jnp.int32),
            pltpu.VMEM_SHARED((_NS, _BPW * _DIM), jnp.float32),
            pltpu.VMEM_SHARED((_NS, _BPW * _DIM), jnp.float32),
            pltpu.VMEM((_CH * _DIM,), jnp.float32),
            pltpu.VMEM((_CH * _DIM,), jnp.float32),
            pltpu.VMEM((_BPW,), jnp.float32),
        ] + [pltpu.SemaphoreType.DMA] * 4,
    )(_sc_body)
    return k(users_vec, items_vec, users_emb, items_emb)


def kernel(users_vec, items_vec, users_emb, items_emb):
    return _run(users_vec, items_vec, users_emb, items_emb)


# speedup vs baseline: 1.0863x; 1.0863x over previous
import functools

import jax
import jax.numpy as jnp
from jax import lax
from jax.experimental import pallas as pl
from jax.experimental.pallas import tpu as pltpu
from jax.experimental.pallas import tpu_sc as plsc

_BATCH = 16384
_DIM = 64
_NC = 2
_NS = 16
_NW = _NC * _NS
_BPW = _BATCH // _NW
_CH = 64
_NCH = _BPW // _CH
_L = 16
_GPC = _CH // _L


def _sc_body(uvec_hbm, ivec_hbm, uemb_hbm, iemb_hbm, out_hbm,
             uixv, iixv, ish, ub0, ub1, ib, outv, *sems):
    cid = lax.axis_index("c")
    sid = lax.axis_index("s")
    wid = sid * _NC + cid
    base = wid * _BPW

    pltpu.sync_copy(uvec_hbm.at[pl.ds(base, _BPW)], uixv)
    pltpu.sync_copy(ivec_hbm.at[pl.ds(base, _BPW)], iixv)

    ubufs = (ub0, ub1)
    usems = (sems[0], sems[1])
    isems = (sems[2], sems[3])

    def fire(c):
        par = c % 2
        ub = ubufs[par]

        def body(q, carry):
            v_u = uixv[pl.ds(c * _CH + q * _L, _L)]
            v_i = iixv[pl.ds(c * _CH + q * _L, _L)]
            for l in range(_L):
                t = q * _L + l
                ti = c * _CH + t
                pltpu.async_copy(uemb_hbm.at[v_u[l]], ub.at[t], usems[par])
                pltpu.async_copy(iemb_hbm.at[v_i[l]],
                                 ish.at[sid, pl.ds(ti * _DIM, _DIM)],
                                 isems[par])
            return carry

        lax.fori_loop(0, _GPC, body, 0)

    cbytes = _CH * _DIM

    def drain(c):
        par = c % 2
        pltpu.make_async_copy(
            uemb_hbm.at[pl.ds(0, _CH)], ubufs[par], usems[par]).wait()
        pltpu.make_async_copy(
            out_hbm.at[pl.ds(0, cbytes)],
            ish.at[sid, pl.ds(0, cbytes)], isems[par]).wait()

    iota = lax.iota(jnp.int32, _L)

    def compute(c):
        par = c % 2
        ub = ubufs[par]
        pltpu.sync_copy(ish.at[sid, pl.ds(c * cbytes, cbytes)], ib)
        for g in range(_GPC):
            rows = g * _L + iota
            rowbase = rows * _DIM

            def body(d, acc):
                dv = jnp.full((_L,), d, jnp.int32)
                uu = plsc.load_gather(ub, [rows, dv])
                vv = plsc.load_gather(ib, [rowbase + d])
                return acc + uu * vv

            acc = lax.fori_loop(0, _DIM, body, jnp.zeros((_L,), jnp.float32))
            outv[pl.ds(c * _CH + g * _L, _L)] = 1.0 / (1.0 + jnp.exp(-acc))

    fire(0)
    for c in range(_NCH):
        if c + 1 < _NCH:
            fire(c + 1)
        drain(c)
        compute(c)

    pltpu.sync_copy(outv, out_hbm.at[pl.ds(base, _BPW)])


@jax.jit
def _run(users_vec, items_vec, users_emb, items_emb):
    k = functools.partial(
        pl.kernel,
        mesh=plsc.VectorSubcoreMesh(core_axis_name="c", subcore_axis_name="s"),
        out_type=jax.ShapeDtypeStruct((_BATCH,), jnp.float32),
        compiler_params=pltpu.CompilerParams(
            needs_layout_passes=False,
        ),
        scratch_types=[
            pltpu.VMEM((_BPW,), jnp.int32),
            pltpu.VMEM((_BPW,), jnp.int32),
            pltpu.VMEM_SHARED((_NS, _BPW * _DIM), jnp.float32),
            pltpu.VMEM((_CH, _DIM), jnp.float32),
            pltpu.VMEM((_CH, _DIM), jnp.float32),
            pltpu.VMEM((_CH * _DIM,), jnp.float32),
            pltpu.VMEM((_BPW,), jnp.float32),
        ] + [pltpu.SemaphoreType.DMA] * 4,
    )(_sc_body)
    return k(users_vec, items_vec, users_emb, items_emb)


def kernel(users_vec, items_vec, users_emb, items_emb):
    return _run(users_vec, items_vec, users_emb, items_emb)


# speedup vs baseline: 1.1471x; 1.0560x over previous
import functools

import jax
import jax.numpy as jnp
from jax import lax
from jax.experimental import pallas as pl
from jax.experimental.pallas import tpu as pltpu
from jax.experimental.pallas import tpu_sc as plsc

_BATCH = 16384
_DIM = 64
_NC = 2
_NS = 16
_NW = _NC * _NS
_BPW = _BATCH // _NW
_CH = 64
_NCH = _BPW // _CH
_L = 16
_GPC = _CH // _L


def _sc_body(uvec_hbm, ivec_hbm, uemb_hbm, iemb_hbm, out_hbm,
             uixv, iixv, ub0, ub1, ib0, ib1, outv,
             sem_u0, sem_u1, sem_i0, sem_i1):
    wid = lax.axis_index("s") * _NC + lax.axis_index("c")
    base = wid * _BPW

    pltpu.sync_copy(uvec_hbm.at[pl.ds(base, _BPW)], uixv)
    pltpu.sync_copy(ivec_hbm.at[pl.ds(base, _BPW)], iixv)

    ubufs = (ub0, ub1)
    ibufs = (ib0, ib1)
    usems = (sem_u0, sem_u1)
    isems = (sem_i0, sem_i1)

    def fire(c):
        par = c % 2
        ub, ib = ubufs[par], ibufs[par]

        def body(q, carry):
            v_u = uixv[pl.ds(c * _CH + q * _L, _L)]
            v_i = iixv[pl.ds(c * _CH + q * _L, _L)]
            for l in range(_L):
                t = q * _L + l
                pltpu.async_copy(uemb_hbm.at[v_u[l]], ub.at[t], usems[par])
                pltpu.async_copy(iemb_hbm.at[v_i[l]], ib.at[t], isems[par])
            return carry

        lax.fori_loop(0, _GPC, body, 0)

    def drain(c):
        par = c % 2
        dummy = uemb_hbm.at[pl.ds(0, _CH)]
        pltpu.make_async_copy(dummy, ubufs[par], usems[par]).wait()
        pltpu.make_async_copy(dummy, ibufs[par], isems[par]).wait()

    iota = lax.iota(jnp.int32, _L)

    def compute(c):
        par = c % 2
        ub, ib = ubufs[par], ibufs[par]
        for g in range(_GPC):
            rows = g * _L + iota

            def body(d, acc):
                dv = jnp.full((_L,), d, jnp.int32)
                uu = plsc.load_gather(ub, [rows, dv])
                vv = plsc.load_gather(ib, [rows, dv])
                return acc + uu * vv

            acc = lax.fori_loop(0, _DIM, body, jnp.zeros((_L,), jnp.float32))
            outv[pl.ds(c * _CH + g * _L, _L)] = 1.0 / (1.0 + jnp.exp(-acc))

    fire(0)
    for c in range(_NCH):
        if c + 1 < _NCH:
            fire(c + 1)
        drain(c)
        compute(c)

    pltpu.sync_copy(outv, out_hbm.at[pl.ds(base, _BPW)])


@jax.jit
def _run(users_vec, items_vec, users_emb, items_emb):
    k = functools.partial(
        pl.kernel,
        mesh=plsc.VectorSubcoreMesh(core_axis_name="c", subcore_axis_name="s"),
        out_type=jax.ShapeDtypeStruct((_BATCH,), jnp.float32),
        compiler_params=pltpu.CompilerParams(
            needs_layout_passes=False,
        ),
        scratch_types=[
            pltpu.VMEM((_BPW,), jnp.int32),
            pltpu.VMEM((_BPW,), jnp.int32),
            pltpu.VMEM((_CH, _DIM), jnp.float32),
            pltpu.VMEM((_CH, _DIM), jnp.float32),
            pltpu.VMEM((_CH, _DIM), jnp.float32),
            pltpu.VMEM((_CH, _DIM), jnp.float32),
            pltpu.VMEM((_BPW,), jnp.float32),
            pltpu.SemaphoreType.DMA,
            pltpu.SemaphoreType.DMA,
            pltpu.SemaphoreType.DMA,
            pltpu.SemaphoreType.DMA,
        ],
    )(_sc_body)
    return k(users_vec, items_vec, users_emb, items_emb)


def kernel(users_vec, items_vec, users_emb, items_emb):
    return _run(users_vec, items_vec, users_emb, items_emb)
